# baseline (speedup 1.0000x reference)
import functools

import numpy as np
import jax
import jax.numpy as jnp
from jax import lax
from jax.experimental import pallas as pl
from jax.experimental.pallas import tpu as pltpu

_B, _N, _P = 2, 4096, 64
_C, _OUT = 64, 64
_NR = 4
_ROWS = 512
_G = 16
_C0 = float(np.sqrt(1.0 / np.pi) / 2.0)
_C1 = float(np.sqrt(3.0 / np.pi) / 2.0)
_C4 = float(np.sqrt(15.0 / np.pi) / 2.0)
_C6 = float(np.sqrt(5.0 / np.pi) / 4.0)
_C8 = float(np.sqrt(15.0 / np.pi) / 4.0)
_INV2SIG2 = 18.0
_RGRID = [0.0, 0.5 / 3.0, 1.0 / 3.0, 0.5]
_LO0 = 0x38D1B717
_HI0 = 0x40800000
_BITS = 27

_BF = jnp.bfloat16


def _body(xyzT_ref, xyzR_ref, sig_ref, w3_ref, bias_ref, out_ref, yb_ref):
    xa = xyzT_ref[0, 0:1, :]
    ya = xyzT_ref[0, 1:2, :]
    za = xyzT_ref[0, 2:3, :]
    xr = xyzR_ref[0, :, 0:1]
    yr = xyzR_ref[0, :, 1:2]
    zr = xyzR_ref[0, :, 2:3]

    dx = xa - xr
    dy = ya - yr
    dz = za - zr

    cross = jnp.dot(
        xyzR_ref[0].astype(_BF),
        xyzT_ref[0].astype(_BF),
        preferred_element_type=jnp.float32,
    )
    r_rows = xr * xr + yr * yr + zr * zr
    r_all = xa * xa + ya * ya + za * za
    dsq = r_rows - 2.0 * cross + r_all

    t = lax.bitcast_convert_type(dsq, jnp.int32)

    lo = jnp.full((_ROWS, 1), _LO0, jnp.int32)
    hi = jnp.full((_ROWS, 1), _HI0, jnp.int32)
    for _ in range(_BITS):
        mid = lo + lax.shift_right_arithmetic(hi - lo, 1)
        cnt = jnp.sum((t <= mid).astype(jnp.int32), axis=1, keepdims=True)
        ge = cnt >= _P
        lo = jnp.where(ge, lo, mid)
        hi = jnp.where(ge, mid, hi)
    kth = hi
    mask = (t <= kth).astype(jnp.float32)

    dsqc = jnp.maximum(dsq, 1e-4)
    rad0 = jnp.exp(dsqc * (-_INV2SIG2))
    e1 = jnp.exp(jnp.sqrt(dsqc) * 6.0)
    y_w = _C0 * jnp.sum(mask * rad0, axis=1, keepdims=True)
    a = mask * (1.0 / (y_w + 1e-6))

    dsq_true = dx * dx + dy * dy + dz * dz
    inv_n = jnp.where(dsq_true >= 1e-4, lax.rsqrt(dsq_true), 100.0)
    ux = dx * inv_n
    uy = dy * inv_n
    uz = dz * inv_n

    sig_bf = sig_ref[0].astype(_BF)

    for g0 in range(0, _ROWS, _G):
        gs = slice(g0, g0 + _G)
        uxg = ux[gs].astype(_BF)
        uyg = uy[gs].astype(_BF)
        uzg = uz[gs].astype(_BF)
        x2, y2, z2 = uxg * uxg, uyg * uyg, uzg * uzg
        ys = [
            jnp.full((_G, _N), _C0, _BF),
            _BF(_C1) * uyg, _BF(_C1) * uzg, _BF(-_C1) * uxg,
            _BF(_C4) * (uxg * uyg), _BF(_C4) * (uyg * uzg),
            _BF(_C6) * (_BF(2.0) * z2 - x2 - y2),
            _BF(-_C4) * (uzg * uxg),
            _BF(_C8) * (x2 - y2),
        ]
        e1g = e1[gs]
        arf = a[gs] * rad0[gs]
        rows = []
        for r in range(_NR):
            if r:
                arf = arf * e1g * float(np.exp(0.5 - r))
            ar = arf.astype(_BF)
            for m in range(9):
                rows.append(ar * ys[m])
        kmat = jnp.concatenate(rows, axis=0)
        y = jnp.dot(kmat, sig_bf, preferred_element_type=jnp.float32)
        y = y * y
        for r in range(_NR):
            b = r * 9 * _G
            b0 = y[b:b + _G]
            b1 = y[b + _G:b + 2 * _G] + y[b + 2 * _G:b + 3 * _G] + y[b + 3 * _G:b + 4 * _G]
            b2 = (y[b + 4 * _G:b + 5 * _G] + y[b + 5 * _G:b + 6 * _G]
                  + y[b + 6 * _G:b + 7 * _G] + y[b + 7 * _G:b + 8 * _G]
                  + y[b + 8 * _G:b + 9 * _G])
            k0 = r * 3
            yb_ref[gs, k0 * _C:(k0 + 1) * _C] = jnp.sqrt(jnp.maximum(b0, 1e-4))
            yb_ref[gs, (k0 + 1) * _C:(k0 + 2) * _C] = jnp.sqrt(jnp.maximum(b1, 1e-4))
            yb_ref[gs, (k0 + 2) * _C:(k0 + 3) * _C] = jnp.sqrt(jnp.maximum(b2, 1e-4))

    acc = jnp.dot(yb_ref[...], w3_ref[...], preferred_element_type=jnp.float32)
    out_ref[0] = acc + bias_ref[0:1, :]


@jax.jit
def kernel(xyz, signal, weight, biases):
    xyzT = jnp.transpose(xyz, (0, 2, 1))
    w3 = jnp.transpose(weight, (2, 3, 1, 0)).reshape(12 * _C, _OUT)
    bias2 = biases.reshape(1, _OUT)
    grid = (_B, _N // _ROWS)
    return pl.pallas_call(
        _body,
        grid=grid,
        in_specs=[
            pl.BlockSpec((1, 3, _N), lambda b, i: (b, 0, 0)),
            pl.BlockSpec((1, _ROWS, 3), lambda b, i: (b, i, 0)),
            pl.BlockSpec((1, _N, _C), lambda b, i: (b, 0, 0)),
            pl.BlockSpec((12 * _C, _OUT), lambda b, i: (0, 0)),
            pl.BlockSpec((1, _OUT), lambda b, i: (0, 0)),
        ],
        out_specs=pl.BlockSpec((1, _ROWS, _OUT), lambda b, i: (b, i, 0)),
        out_shape=jax.ShapeDtypeStruct((_B, _N, _OUT), jnp.float32),
        scratch_shapes=[pltpu.VMEM((_ROWS, 12 * _C), jnp.float32)],
    )(xyzT, xyz, signal, w3, bias2)


# speedup vs baseline: 1.1634x; 1.1634x over previous
import functools

import numpy as np
import jax
import jax.numpy as jnp
from jax import lax
from jax.experimental import pallas as pl
from jax.experimental.pallas import tpu as pltpu

_B, _N, _P = 2, 4096, 64
_C, _OUT = 64, 64
_NR = 4
_ROWS = 256
_G = 16
_C0 = float(np.sqrt(1.0 / np.pi) / 2.0)
_C1 = float(np.sqrt(3.0 / np.pi) / 2.0)
_C4 = float(np.sqrt(15.0 / np.pi) / 2.0)
_C6 = float(np.sqrt(5.0 / np.pi) / 4.0)
_C8 = float(np.sqrt(15.0 / np.pi) / 4.0)
_INV2SIG2 = 18.0
_RGRID = [0.0, 0.5 / 3.0, 1.0 / 3.0, 0.5]
_LO0 = 0x38D1B717
_HI0 = 0x40800000
_BITS = 27

_BF = jnp.bfloat16


def _body(xyzT_ref, xyzR_ref, sig_ref, w3_ref, bias_ref, out_ref, yb_ref):
    xa = xyzT_ref[0, 0:1, :]
    ya = xyzT_ref[0, 1:2, :]
    za = xyzT_ref[0, 2:3, :]
    xr = xyzR_ref[0, :, 0:1]
    yr = xyzR_ref[0, :, 1:2]
    zr = xyzR_ref[0, :, 2:3]

    dx = xa - xr
    dy = ya - yr
    dz = za - zr

    cross = jnp.dot(
        xyzR_ref[0].astype(_BF),
        xyzT_ref[0].astype(_BF),
        preferred_element_type=jnp.float32,
    )
    r_rows = xr * xr + yr * yr + zr * zr
    r_all = xa * xa + ya * ya + za * za
    dsq = r_rows - 2.0 * cross + r_all

    t = lax.bitcast_convert_type(dsq, jnp.int32)

    lo = jnp.full((_ROWS, 1), _LO0, jnp.int32)
    hi = jnp.full((_ROWS, 1), _HI0, jnp.int32)
    for _ in range(_BITS):
        mid = lo + lax.shift_right_arithmetic(hi - lo, 1)
        cnt = jnp.sum((t <= mid).astype(jnp.int32), axis=1, keepdims=True)
        ge = cnt >= _P
        lo = jnp.where(ge, lo, mid)
        hi = jnp.where(ge, mid, hi)
    kth = hi
    mask = (t <= kth).astype(jnp.float32)

    dsqc = jnp.maximum(dsq, 1e-4)
    rad0 = jnp.exp(dsqc * (-_INV2SIG2))
    e1 = jnp.exp(jnp.sqrt(dsqc) * 6.0)
    y_w = _C0 * jnp.sum(mask * rad0, axis=1, keepdims=True)
    a = mask * (1.0 / (y_w + 1e-6))

    dsq_true = dx * dx + dy * dy + dz * dz
    inv_n = jnp.where(dsq_true >= 1e-4, lax.rsqrt(dsq_true), 100.0)
    ux = dx * inv_n
    uy = dy * inv_n
    uz = dz * inv_n

    sig_bf = sig_ref[0].astype(_BF)

    for g0 in range(0, _ROWS, _G):
        gs = slice(g0, g0 + _G)
        uxg = ux[gs].astype(_BF)
        uyg = uy[gs].astype(_BF)
        uzg = uz[gs].astype(_BF)
        x2, y2, z2 = uxg * uxg, uyg * uyg, uzg * uzg
        ys = [
            jnp.full((_G, _N), _C0, _BF),
            _BF(_C1) * uyg, _BF(_C1) * uzg, _BF(-_C1) * uxg,
            _BF(_C4) * (uxg * uyg), _BF(_C4) * (uyg * uzg),
            _BF(_C6) * (_BF(2.0) * z2 - x2 - y2),
            _BF(-_C4) * (uzg * uxg),
            _BF(_C8) * (x2 - y2),
        ]
        e1g = e1[gs]
        arf = a[gs] * rad0[gs]
        rows = []
        for r in range(_NR):
            if r:
                arf = arf * e1g * float(np.exp(0.5 - r))
            ar = arf.astype(_BF)
            for m in range(9):
                rows.append(ar * ys[m])
        kmat = jnp.concatenate(rows, axis=0)
        y = jnp.dot(kmat, sig_bf, preferred_element_type=jnp.float32)
        y = y * y
        for r in range(_NR):
            b = r * 9 * _G
            b0 = y[b:b + _G]
            b1 = y[b + _G:b + 2 * _G] + y[b + 2 * _G:b + 3 * _G] + y[b + 3 * _G:b + 4 * _G]
            b2 = (y[b + 4 * _G:b + 5 * _G] + y[b + 5 * _G:b + 6 * _G]
                  + y[b + 6 * _G:b + 7 * _G] + y[b + 7 * _G:b + 8 * _G]
                  + y[b + 8 * _G:b + 9 * _G])
            k0 = r * 3
            yb_ref[gs, k0 * _C:(k0 + 1) * _C] = jnp.sqrt(jnp.maximum(b0, 1e-4))
            yb_ref[gs, (k0 + 1) * _C:(k0 + 2) * _C] = jnp.sqrt(jnp.maximum(b1, 1e-4))
            yb_ref[gs, (k0 + 2) * _C:(k0 + 3) * _C] = jnp.sqrt(jnp.maximum(b2, 1e-4))

    acc = jnp.dot(yb_ref[...], w3_ref[...], preferred_element_type=jnp.float32)
    out_ref[0] = acc + bias_ref[0:1, :]


@jax.jit
def kernel(xyz, signal, weight, biases):
    xyzT = jnp.transpose(xyz, (0, 2, 1))
    w3 = jnp.transpose(weight, (2, 3, 1, 0)).reshape(12 * _C, _OUT)
    bias2 = biases.reshape(1, _OUT)
    grid = (_B, _N // _ROWS)
    return pl.pallas_call(
        _body,
        grid=grid,
        in_specs=[
            pl.BlockSpec((1, 3, _N), lambda b, i: (b, 0, 0)),
            pl.BlockSpec((1, _ROWS, 3), lambda b, i: (b, i, 0)),
            pl.BlockSpec((1, _N, _C), lambda b, i: (b, 0, 0)),
            pl.BlockSpec((12 * _C, _OUT), lambda b, i: (0, 0)),
            pl.BlockSpec((1, _OUT), lambda b, i: (0, 0)),
        ],
        out_specs=pl.BlockSpec((1, _ROWS, _OUT), lambda b, i: (b, i, 0)),
        out_shape=jax.ShapeDtypeStruct((_B, _N, _OUT), jnp.float32),
        scratch_shapes=[pltpu.VMEM((_ROWS, 12 * _C), jnp.float32)],
    )(xyzT, xyz, signal, w3, bias2)
